# baseline (speedup 1.0000x reference)
import functools

import jax
import jax.numpy as jnp
from jax import lax
from jax.experimental import pallas as pl
from jax.experimental.pallas import tpu as pltpu
from jax.experimental.pallas import tpu_sc as plsc

NUM_EMBEDDINGS = 1000000
EMBEDDING_DIM = 64
BATCH = 16384
HIST_LEN = 20
B_TOTAL = BATCH * HIST_LEN

NUM_CORES = 2
NUM_SUBCORES = 16
J_PER_CORE = EMBEDDING_DIM // NUM_CORES
B_PER_TILE = BATCH // NUM_SUBCORES
ELEMS_PER_TILE = HIST_LEN * B_PER_TILE
N_GSTREAMS = 10
G_CHUNK = ELEMS_PER_TILE // N_GSTREAMS


@functools.lru_cache(maxsize=1)
def _build_planf():
    mesh = plsc.VectorSubcoreMesh(core_axis_name="c", subcore_axis_name="s")

    @functools.partial(
        pl.kernel,
        mesh=mesh,
        compiler_params=pltpu.CompilerParams(use_tc_tiling_on_sc=True),
        out_type=jax.ShapeDtypeStruct((HIST_LEN, EMBEDDING_DIM, BATCH), jnp.float32),
        scratch_types=(
            [pltpu.VMEM_SHARED((NUM_EMBEDDINGS,), jnp.float32)]
            + [pltpu.VMEM((ELEMS_PER_TILE,), jnp.int32)]
            + [pltpu.VMEM((ELEMS_PER_TILE,), jnp.float32)] * 2
            + [pltpu.SemaphoreType.DMA] * (1 + N_GSTREAMS + 2)
        ),
    )
    def _planf(wt_hbm, idx_hbm, out_hbm, row_sp, idx_v, val0, val1, *sems):
        rsem = sems[0]
        gsems = sems[1 : 1 + N_GSTREAMS]
        wsems = sems[1 + N_GSTREAMS :]
        c = lax.axis_index("c")
        s = lax.axis_index("s")
        vals = (val0, val1)

        def row_copy(jj):
            j = c * J_PER_CORE + jj
            return pltpu.make_async_copy(wt_hbm.at[j], row_sp, rsem)

        H_PER_STREAM = HIST_LEN // N_GSTREAMS

        def gather(jj):
            hs = []
            for g in range(N_GSTREAMS):
                hs.append(
                    pltpu.async_copy(
                        row_sp.at[idx_v.at[pl.ds(g * G_CHUNK, G_CHUNK)]],
                        vals[jj % 2].at[pl.ds(g * G_CHUNK, G_CHUNK)],
                        gsems[g],
                    )
                )
            return hs

        def write_h(jj, h):
            j = c * J_PER_CORE + jj
            return pltpu.async_copy(
                vals[jj % 2].at[pl.ds(h * B_PER_TILE, B_PER_TILE)],
                out_hbm.at[h, j, pl.ds(s * B_PER_TILE, B_PER_TILE)],
                wsems[jj % 2],
            )

        write_handles = [None] * J_PER_CORE

        first_copy = row_copy(0)

        @pl.when(s == 0)
        def _():
            first_copy.start()

        stage = [
            pltpu.make_async_copy(
                idx_hbm.at[pl.ds(h * BATCH + s * B_PER_TILE, B_PER_TILE)],
                idx_v.at[pl.ds(h * B_PER_TILE, B_PER_TILE)],
                wsems[0],
            )
            for h in range(HIST_LEN)
        ]
        for cp in stage:
            cp.start()
        for cp in stage:
            cp.wait()

        for jj in range(J_PER_CORE):
            cur_copy = row_copy(jj)

            @pl.when(s == 0)
            def _():
                cur_copy.wait()
            if jj >= 2:
                for hnd in write_handles[jj - 2]:
                    hnd.wait()
            plsc.subcore_barrier()
            ghs = gather(jj)
            whs = []
            for g in range(N_GSTREAMS):
                ghs[g].wait()
                for h in range(g * H_PER_STREAM, (g + 1) * H_PER_STREAM):
                    whs.append(write_h(jj, h))
            plsc.subcore_barrier()

            if jj + 1 < J_PER_CORE:
                next_copy = row_copy(jj + 1)

                @pl.when(s == 0)
                def _():
                    next_copy.start()

            write_handles[jj] = whs

        for jj in (J_PER_CORE - 2, J_PER_CORE - 1):
            for hnd in write_handles[jj]:
                hnd.wait()

    return _planf


def kernel(input_, weight):
    wt = weight.T
    idxt = input_.T.astype(jnp.int32).reshape((B_TOTAL,))
    out_t = _build_planf()(wt, idxt)
    return out_t.transpose(2, 0, 1)


# speedup vs baseline: 1.0044x; 1.0044x over previous
import functools

import jax
import jax.numpy as jnp
from jax import lax
from jax.experimental import pallas as pl
from jax.experimental.pallas import tpu as pltpu
from jax.experimental.pallas import tpu_sc as plsc

NUM_EMBEDDINGS = 1000000
EMBEDDING_DIM = 64
BATCH = 16384
HIST_LEN = 20
B_TOTAL = BATCH * HIST_LEN

NUM_CORES = 2
NUM_SUBCORES = 16
J_PER_CORE = EMBEDDING_DIM // NUM_CORES
B_PER_TILE = BATCH // NUM_SUBCORES
ELEMS_PER_TILE = HIST_LEN * B_PER_TILE
N_GSTREAMS = 4
G_CHUNK = ELEMS_PER_TILE // N_GSTREAMS


@functools.lru_cache(maxsize=1)
def _build_planf():
    mesh = plsc.VectorSubcoreMesh(core_axis_name="c", subcore_axis_name="s")

    @functools.partial(
        pl.kernel,
        mesh=mesh,
        compiler_params=pltpu.CompilerParams(use_tc_tiling_on_sc=True),
        out_type=jax.ShapeDtypeStruct((HIST_LEN, EMBEDDING_DIM, BATCH), jnp.float32),
        scratch_types=(
            [pltpu.VMEM_SHARED((NUM_EMBEDDINGS,), jnp.float32)]
            + [pltpu.VMEM((ELEMS_PER_TILE,), jnp.int32)]
            + [pltpu.VMEM((ELEMS_PER_TILE,), jnp.float32)] * 2
            + [pltpu.SemaphoreType.DMA] * (1 + N_GSTREAMS + 2)
        ),
    )
    def _planf(wt_hbm, idx_hbm, out_hbm, row_sp, idx_v, val0, val1, *sems):
        rsem = sems[0]
        gsems = sems[1 : 1 + N_GSTREAMS]
        wsems = sems[1 + N_GSTREAMS :]
        c = lax.axis_index("c")
        s = lax.axis_index("s")
        vals = (val0, val1)

        def row_copy(jj):
            j = c * J_PER_CORE + jj
            return pltpu.make_async_copy(wt_hbm.at[j], row_sp, rsem)

        H_PER_STREAM = HIST_LEN // N_GSTREAMS

        def gather(jj):
            hs = []
            for g in range(N_GSTREAMS):
                hs.append(
                    pltpu.async_copy(
                        row_sp.at[idx_v.at[pl.ds(g * G_CHUNK, G_CHUNK)]],
                        vals[jj % 2].at[pl.ds(g * G_CHUNK, G_CHUNK)],
                        gsems[g],
                    )
                )
            return hs

        def write_h(jj, h):
            j = c * J_PER_CORE + jj
            return pltpu.async_copy(
                vals[jj % 2].at[pl.ds(h * B_PER_TILE, B_PER_TILE)],
                out_hbm.at[h, j, pl.ds(s * B_PER_TILE, B_PER_TILE)],
                wsems[jj % 2],
            )

        write_handles = [None] * J_PER_CORE

        first_copy = row_copy(0)

        @pl.when(s == 0)
        def _():
            first_copy.start()

        stage = [
            pltpu.make_async_copy(
                idx_hbm.at[pl.ds(h * BATCH + s * B_PER_TILE, B_PER_TILE)],
                idx_v.at[pl.ds(h * B_PER_TILE, B_PER_TILE)],
                wsems[0],
            )
            for h in range(HIST_LEN)
        ]
        for cp in stage:
            cp.start()
        for cp in stage:
            cp.wait()

        for jj in range(J_PER_CORE):
            cur_copy = row_copy(jj)

            @pl.when(s == 0)
            def _():
                cur_copy.wait()
            if jj >= 2:
                for hnd in write_handles[jj - 2]:
                    hnd.wait()
            plsc.subcore_barrier()
            ghs = gather(jj)
            whs = []
            for g in range(N_GSTREAMS):
                ghs[g].wait()
                for h in range(g * H_PER_STREAM, (g + 1) * H_PER_STREAM):
                    whs.append(write_h(jj, h))
            plsc.subcore_barrier()

            if jj + 1 < J_PER_CORE:
                next_copy = row_copy(jj + 1)

                @pl.when(s == 0)
                def _():
                    next_copy.start()

            write_handles[jj] = whs

        for jj in (J_PER_CORE - 2, J_PER_CORE - 1):
            for hnd in write_handles[jj]:
                hnd.wait()

    return _planf


def kernel(input_, weight):
    wt = weight.T
    idxt = input_.T.astype(jnp.int32).reshape((B_TOTAL,))
    out_t = _build_planf()(wt, idxt)
    return out_t.transpose(2, 0, 1)
